# Initial kernel scaffold; baseline (speedup 1.0000x reference)
import jax
import jax.numpy as jnp
from jax.experimental import pallas as pl


def kernel(x, x_batch, tgt_y, tgt_edge_index, tgt_edge_type, tgt_y_batch, emb, Wmsg, Wself, Winit, Et, Wq, Wk, Wv, Wnode, Wedge):
    raise NotImplementedError("write your pallas kernel here")


# speedup vs baseline: 2.6062x; 2.6062x over previous
import functools
import math

import jax
import jax.numpy as jnp
from jax import lax
from jax.experimental import pallas as pl
from jax.experimental.pallas import tpu as pltpu
from jax.experimental.pallas import tpu_sc as plsc

_N = 10000
_E = 320000
_NX = 1024
_D = 128
_V = 512
_NC = 2
_NS = 16
_NW = _NC * _NS
_EPW = _E // _NW
_CH = 80
_NCHUNK = _EPW // _CH
_NP = 10112
_RPW = _NP // _NS
_BN = 400
_GRID = _N // _BN

_mesh = plsc.VectorSubcoreMesh(core_axis_name="c", subcore_axis_name="s")


def _wid():
    return lax.axis_index("s") * _NC + lax.axis_index("c")


def _counts_body(dst_hbm, et_hbm, id16_hbm, zeros16_hbm, out_hbm,
                 didx, etv, oh, cnt_sh, sem):
    c = lax.axis_index("c")
    s = lax.axis_index("s")
    w = _wid()
    pltpu.sync_copy(zeros16_hbm.at[pl.ds(s * _RPW, _RPW)],
                    cnt_sh.at[pl.ds(s * _RPW, _RPW)])
    plsc.subcore_barrier()

    def chunk(i, carry):
        base = w * _EPW + i * _CH
        pltpu.sync_copy(dst_hbm.at[pl.ds(base, _CH)], didx)
        pltpu.sync_copy(et_hbm.at[pl.ds(base, _CH)], etv)
        pltpu.async_copy(id16_hbm.at[etv], oh, sem).wait()
        pltpu.sync_copy(oh, cnt_sh.at[didx], add=True)
        return carry

    lax.fori_loop(0, _NCHUNK, chunk, 0)
    plsc.subcore_barrier()
    pltpu.sync_copy(cnt_sh.at[pl.ds(s * _RPW, _RPW)],
                    out_hbm.at[c, pl.ds(s * _RPW, _RPW)])


_counts_call = pl.kernel(
    _counts_body,
    out_type=jax.ShapeDtypeStruct((_NC, _NP, 16), jnp.float32),
    mesh=_mesh,
    compiler_params=pltpu.CompilerParams(use_tc_tiling_on_sc=False),
    scratch_types=[
        pltpu.VMEM((_CH,), jnp.int32),
        pltpu.VMEM((_CH,), jnp.int32),
        pltpu.VMEM((_CH, 16), jnp.float32),
        pltpu.VMEM_SHARED((_NP, 16), jnp.float32),
        pltpu.SemaphoreType.DMA,
    ],
)


def _spmm_body(m_hbm, src_hbm, dst_hbm, zeros_hbm, out_hbm,
               sidx, didx, rows, agg_sh, sem):
    c = lax.axis_index("c")
    s = lax.axis_index("s")
    w = _wid()
    pltpu.sync_copy(zeros_hbm.at[pl.ds(s * _RPW, _RPW)],
                    agg_sh.at[pl.ds(s * _RPW, _RPW)])
    plsc.subcore_barrier()

    def chunk(i, carry):
        base = w * _EPW + i * _CH
        pltpu.sync_copy(src_hbm.at[pl.ds(base, _CH)], sidx)
        pltpu.sync_copy(dst_hbm.at[pl.ds(base, _CH)], didx)
        pltpu.async_copy(m_hbm.at[sidx], rows, sem).wait()
        pltpu.sync_copy(rows, agg_sh.at[didx], add=True)
        return carry

    lax.fori_loop(0, _NCHUNK, chunk, 0)
    plsc.subcore_barrier()
    pltpu.sync_copy(agg_sh.at[pl.ds(s * _RPW, _RPW)],
                    out_hbm.at[c, pl.ds(s * _RPW, _RPW)])


_spmm_call = pl.kernel(
    _spmm_body,
    out_type=jax.ShapeDtypeStruct((_NC, _NP, _D), jnp.float32),
    mesh=_mesh,
    scratch_types=[
        pltpu.VMEM((_CH,), jnp.int32),
        pltpu.VMEM((_CH,), jnp.int32),
        pltpu.VMEM((_CH, _D), jnp.float32),
        pltpu.VMEM_SHARED((_NP, _D), jnp.float32),
        pltpu.SemaphoreType.DMA,
    ],
)


def _edge_body(es_hbm, ed_hbm, src_hbm, dst_hbm, out_hbm,
               sidx, didx, esr, edr, sem):
    w = _wid()

    def chunk(i, carry):
        base = w * _EPW + i * _CH
        pltpu.sync_copy(src_hbm.at[pl.ds(base, _CH)], sidx)
        pltpu.sync_copy(dst_hbm.at[pl.ds(base, _CH)], didx)
        pltpu.async_copy(es_hbm.at[sidx], esr, sem).wait()
        pltpu.async_copy(ed_hbm.at[didx], edr, sem).wait()
        for r in range(_CH):
            esr[r, :] = esr[r, :] + edr[r, :]
        pltpu.sync_copy(esr, out_hbm.at[pl.ds(base, _CH)])
        return carry

    lax.fori_loop(0, _NCHUNK, chunk, 0)


_edge_call = pl.kernel(
    _edge_body,
    out_type=jax.ShapeDtypeStruct((_E, 16), jnp.float32),
    mesh=_mesh,
    compiler_params=pltpu.CompilerParams(use_tc_tiling_on_sc=False),
    scratch_types=[
        pltpu.VMEM((_CH,), jnp.int32),
        pltpu.VMEM((_CH,), jnp.int32),
        pltpu.VMEM((_CH, 16), jnp.float32),
        pltpu.VMEM((_CH, 16), jnp.float32),
        pltpu.SemaphoreType.DMA,
    ],
)


def _embed_body(ty_ref, emb_ref, wm0_ref, yinit_ref, m0_ref):
    ty = ty_ref[0, 0, :]
    oh = (ty[:, None] == lax.broadcasted_iota(jnp.int32, (_BN, _V), 1))
    oh = oh.astype(jnp.float32)
    yi = jnp.dot(oh, emb_ref[...], preferred_element_type=jnp.float32, precision=lax.Precision.HIGHEST)
    yinit_ref[...] = yi
    m0_ref[...] = jnp.dot(yi, wm0_ref[...], preferred_element_type=jnp.float32)


def _embed(ty3, emb, wm0):
    return pl.pallas_call(
        _embed_body,
        grid=(_GRID,),
        in_specs=[
            pl.BlockSpec((1, 1, _BN), lambda i: (i, 0, 0)),
            pl.BlockSpec((_V, _D), lambda i: (0, 0)),
            pl.BlockSpec((_D, _D), lambda i: (0, 0)),
        ],
        out_specs=[
            pl.BlockSpec((_BN, _D), lambda i: (i, 0)),
            pl.BlockSpec((_BN, _D), lambda i: (i, 0)),
        ],
        out_shape=[
            jax.ShapeDtypeStruct((_N, _D), jnp.float32),
            jax.ShapeDtypeStruct((_N, _D), jnp.float32),
        ],
    )(ty3, emb, wm0)


def _kv_body(x_ref, w_ref, o_ref):
    o_ref[0] = jnp.dot(x_ref[...], w_ref[0], preferred_element_type=jnp.float32)


def _kv(x, wkv):
    return pl.pallas_call(
        _kv_body,
        grid=(6,),
        in_specs=[
            pl.BlockSpec((_NX, _D), lambda i: (0, 0)),
            pl.BlockSpec((1, _D, _D), lambda i: (i, 0, 0)),
        ],
        out_specs=pl.BlockSpec((1, _NX, _D), lambda i: (i, 0, 0)),
        out_shape=jax.ShapeDtypeStruct((6, _NX, _D), jnp.float32),
    )(x, wkv)


def _block_body(agg_ref, y_ref, yi_ref, zc_ref, ws_ref, wi_ref, wet_ref,
                wq_ref, k_ref, v_ref, wp_ref, yo_ref, alpha_ref, post_ref):
    h = (agg_ref[0] + agg_ref[1]
         + jnp.dot(y_ref[...], ws_ref[...], preferred_element_type=jnp.float32)
         + jnp.dot(yi_ref[...], wi_ref[...], preferred_element_type=jnp.float32)
         + jnp.dot(zc_ref[...], wet_ref[...], preferred_element_type=jnp.float32,
                   precision=lax.Precision.HIGHEST))
    q = jnp.dot(h, wq_ref[...], preferred_element_type=jnp.float32)
    sc = lax.dot_general(q, k_ref[...], (((1,), (1,)), ((), ())),
                         preferred_element_type=jnp.float32)
    sc = sc * (1.0 / math.sqrt(_D))
    mx = jnp.max(sc, axis=1, keepdims=True)
    e = jnp.exp(sc - mx)
    a = e / jnp.sum(e, axis=1, keepdims=True)
    ctx = jnp.dot(a, v_ref[...], preferred_element_type=jnp.float32)
    yo = jnp.maximum(h + ctx, 0.0)
    yo_ref[...] = yo
    alpha_ref[...] = a
    post_ref[...] = jnp.dot(yo, wp_ref[...], preferred_element_type=jnp.float32)


def _block(aggp, y, y_init, zc, ws, wi, wet, wq, k, v, wpost):
    pd = wpost.shape[1]
    return pl.pallas_call(
        _block_body,
        grid=(_GRID,),
        in_specs=[
            pl.BlockSpec((_NC, _BN, _D), lambda i: (0, i, 0)),
            pl.BlockSpec((_BN, _D), lambda i: (i, 0)),
            pl.BlockSpec((_BN, _D), lambda i: (i, 0)),
            pl.BlockSpec((_BN, _D), lambda i: (i, 0)),
            pl.BlockSpec((_D, _D), lambda i: (0, 0)),
            pl.BlockSpec((_D, _D), lambda i: (0, 0)),
            pl.BlockSpec((_D, _D), lambda i: (0, 0)),
            pl.BlockSpec((_D, _D), lambda i: (0, 0)),
            pl.BlockSpec((_NX, _D), lambda i: (0, 0)),
            pl.BlockSpec((_NX, _D), lambda i: (0, 0)),
            pl.BlockSpec((_D, pd), lambda i: (0, 0)),
        ],
        out_specs=[
            pl.BlockSpec((_BN, _D), lambda i: (i, 0)),
            pl.BlockSpec((_BN, _NX), lambda i: (i, 0)),
            pl.BlockSpec((_BN, pd), lambda i: (i, 0)),
        ],
        out_shape=[
            jax.ShapeDtypeStruct((_N, _D), jnp.float32),
            jax.ShapeDtypeStruct((_N, _NX), jnp.float32),
            jax.ShapeDtypeStruct((_N, pd), jnp.float32),
        ],
    )(aggp, y, y_init, zc, ws, wi, wet, wq, k, v, wpost)


def kernel(x, x_batch, tgt_y, tgt_edge_index, tgt_edge_type, tgt_y_batch,
           emb, Wmsg, Wself, Winit, Et, Wq, Wk, Wv, Wnode, Wedge):
    f32 = jnp.float32
    src = tgt_edge_index[0]
    dst = tgt_edge_index[1]
    zeros16 = jnp.zeros((_NP, 16), f32)
    zerosD = jnp.zeros((_NP, _D), f32)

    cparts = _counts_call(dst, tgt_edge_type, jnp.eye(16, dtype=f32), zeros16)
    counts16 = (cparts[0] + cparts[1])[:_N]

    ty3 = tgt_y.reshape(_GRID, 1, _BN)
    y_init, m = _embed(ty3, emb, Wmsg[0])

    kv = _kv(x, jnp.concatenate([Wk, Wv], axis=0))

    zc = jnp.concatenate([counts16, jnp.zeros((_N, 112), f32)], axis=1)

    wpost2 = jnp.concatenate([
        Wnode,
        Wedge[:_D], jnp.zeros((_D, 8), f32),
        Wedge[_D:], jnp.zeros((_D, 8), f32),
        jnp.zeros((_D, 96), f32),
    ], axis=1)

    alphas = []
    y = y_init
    for i in range(3):
        wet = jnp.concatenate([Et[i], jnp.zeros((124, _D), f32)], axis=0)
        aggp = _spmm_call(m, src, dst, zerosD)
        wpost = wpost2 if i == 2 else Wmsg[i + 1]
        y, a, post = _block(aggp, y, y_init, zc, Wself[i], Winit[i], wet,
                            Wq[i], kv[i], kv[3 + i], wpost)
        alphas.append(a)
        m = post if i < 2 else None

    y_score = post[:, :_V]
    es16 = post[:, _V:_V + 16]
    ed16 = post[:, _V + 16:_V + 32]
    edge16 = _edge_call(es16, ed16, src, dst)
    y_edge_rel_score = edge16[:, :8]

    return (y_score, y_edge_rel_score, alphas[0], alphas[1], alphas[2])


# speedup vs baseline: 3.6150x; 1.3871x over previous
import functools
import math

import jax
import jax.numpy as jnp
from jax import lax
from jax.experimental import pallas as pl
from jax.experimental.pallas import tpu as pltpu
from jax.experimental.pallas import tpu_sc as plsc

_N = 10000
_E = 320000
_NX = 1024
_D = 128
_V = 512
_NC = 2
_NS = 16
_NW = _NC * _NS
_EPW = _E // _NW
_CH = 125
_NCHUNK = _EPW // _CH
_NB = 4
_NP = 10112
_RPW = _NP // _NS
_BN = 400
_GRID = _N // _BN

_mesh = plsc.VectorSubcoreMesh(core_axis_name="c", subcore_axis_name="s")


def _wid():
    return lax.axis_index("s") * _NC + lax.axis_index("c")


def _counts_body(dst2_hbm, et2_hbm, id16_hbm, zeros16_hbm, out_hbm,
                 didx2, etv2, oh0, oh1, oh2, oh3, cnt_sh, g0, g1, g2, g3):
    c = lax.axis_index("c")
    s = lax.axis_index("s")
    w = _wid()
    pltpu.sync_copy(zeros16_hbm.at[pl.ds(s * _RPW, _RPW)],
                    cnt_sh.at[pl.ds(s * _RPW, _RPW)])
    pltpu.sync_copy(dst2_hbm.at[pl.ds(w * _NCHUNK, _NCHUNK)], didx2)
    pltpu.sync_copy(et2_hbm.at[pl.ds(w * _NCHUNK, _NCHUNK)], etv2)
    plsc.subcore_barrier()
    bufs = (oh0, oh1, oh2, oh3)
    sems = (g0, g1, g2, g3)
    for b in range(_NB):
        pltpu.async_copy(id16_hbm.at[etv2.at[b]], bufs[b], sems[b])

    def body(i, carry):
        for b in range(_NB):
            j = i * _NB + b
            pltpu.make_async_copy(id16_hbm.at[etv2.at[j]], bufs[b], sems[b]).wait()
            pltpu.sync_copy(bufs[b], cnt_sh.at[didx2.at[j]], add=True)

            @pl.when(j + _NB < _NCHUNK)
            def _():
                pltpu.async_copy(id16_hbm.at[etv2.at[j + _NB]], bufs[b], sems[b])
        return carry

    lax.fori_loop(0, _NCHUNK // _NB, body, 0)
    plsc.subcore_barrier()
    pltpu.sync_copy(cnt_sh.at[pl.ds(s * _RPW, _RPW)],
                    out_hbm.at[c, pl.ds(s * _RPW, _RPW)])


_counts_call = pl.kernel(
    _counts_body,
    out_type=jax.ShapeDtypeStruct((_NC, _NP, 16), jnp.float32),
    mesh=_mesh,
    compiler_params=pltpu.CompilerParams(use_tc_tiling_on_sc=False),
    scratch_types=[
        pltpu.VMEM((_NCHUNK, _CH), jnp.int32),
        pltpu.VMEM((_NCHUNK, _CH), jnp.int32),
        pltpu.VMEM((_CH, 16), jnp.float32),
        pltpu.VMEM((_CH, 16), jnp.float32),
        pltpu.VMEM((_CH, 16), jnp.float32),
        pltpu.VMEM((_CH, 16), jnp.float32),
        pltpu.VMEM_SHARED((_NP, 16), jnp.float32),
        pltpu.SemaphoreType.DMA,
        pltpu.SemaphoreType.DMA,
        pltpu.SemaphoreType.DMA,
        pltpu.SemaphoreType.DMA,
    ],
)


def _spmm_body(m_hbm, src2_hbm, dst2_hbm, zeros_hbm, out_hbm,
               si0, si1, si2, si3, di0, di1, di2, di3, r0, r1, agg_sh,
               is0, is1, is2, is3, id0, id1, id2, id3, g0, g1):
    c = lax.axis_index("c")
    s = lax.axis_index("s")
    w = _wid()
    pltpu.sync_copy(zeros_hbm.at[pl.ds(s * _RPW, _RPW)],
                    agg_sh.at[pl.ds(s * _RPW, _RPW)])
    plsc.subcore_barrier()
    sib = (si0, si1, si2, si3)
    dib = (di0, di1, di2, di3)
    isem = (is0, is1, is2, is3)
    idsem = (id0, id1, id2, id3)
    rows = (r0, r1)
    gsem = (g0, g1)
    cb = w * _NCHUNK

    def idx_start(j, sl):
        pltpu.async_copy(src2_hbm.at[cb + j], sib[sl], isem[sl])
        pltpu.async_copy(dst2_hbm.at[cb + j], dib[sl], idsem[sl])

    def idx_wait(j, sl):
        pltpu.make_async_copy(src2_hbm.at[cb + j], sib[sl], isem[sl]).wait()
        pltpu.make_async_copy(dst2_hbm.at[cb + j], dib[sl], idsem[sl]).wait()

    for sl in range(4):
        idx_start(sl, sl)
    idx_wait(0, 0)
    idx_wait(1, 1)
    pltpu.async_copy(m_hbm.at[sib[0]], rows[0], gsem[0])
    pltpu.async_copy(m_hbm.at[sib[1]], rows[1], gsem[1])

    def body(i, carry):
        for b4 in range(4):
            j = i * 4 + b4
            b2 = b4 % 2

            pltpu.make_async_copy(m_hbm.at[sib[b4]], rows[b2], gsem[b2]).wait()
            pltpu.sync_copy(rows[b2], agg_sh.at[dib[b4]], add=True)

            @pl.when(j + 4 < _NCHUNK)
            def _():
                idx_start(j + 4, b4)

            @pl.when(j + 2 < _NCHUNK)
            def _():
                sl = (b4 + 2) % 4
                idx_wait(j + 2, sl)
                pltpu.async_copy(m_hbm.at[sib[sl]], rows[b2], gsem[b2])
        return carry

    lax.fori_loop(0, _NCHUNK // 4, body, 0)
    plsc.subcore_barrier()
    pltpu.sync_copy(agg_sh.at[pl.ds(s * _RPW, _RPW)],
                    out_hbm.at[c, pl.ds(s * _RPW, _RPW)])


_spmm_call = pl.kernel(
    _spmm_body,
    out_type=jax.ShapeDtypeStruct((_NC, _NP, _D), jnp.float32),
    mesh=_mesh,
    scratch_types=(
        [pltpu.VMEM((_CH,), jnp.int32)] * 8
        + [pltpu.VMEM((_CH, _D), jnp.float32)] * 2
        + [pltpu.VMEM_SHARED((_NP, _D), jnp.float32)]
        + [pltpu.SemaphoreType.DMA] * 10
    ),
)


def _edge_body(es_hbm, ed_hbm, src2_hbm, dst2_hbm, out_hbm,
               sidx2, didx2, es0, es1, es2, es3, ed0, ed1, ed2, ed3,
               gs0, gs1, gs2, gs3, gd0, gd1, gd2, gd3):
    w = _wid()
    pltpu.sync_copy(src2_hbm.at[pl.ds(w * _NCHUNK, _NCHUNK)], sidx2)
    pltpu.sync_copy(dst2_hbm.at[pl.ds(w * _NCHUNK, _NCHUNK)], didx2)
    esb = (es0, es1, es2, es3)
    edb = (ed0, ed1, ed2, ed3)
    gss = (gs0, gs1, gs2, gs3)
    gds = (gd0, gd1, gd2, gd3)
    for b in range(_NB):
        pltpu.async_copy(es_hbm.at[sidx2.at[b]], esb[b], gss[b])
        pltpu.async_copy(ed_hbm.at[didx2.at[b]], edb[b], gds[b])

    def body(i, carry):
        for b in range(_NB):
            j = i * _NB + b
            pltpu.make_async_copy(es_hbm.at[sidx2.at[j]], esb[b], gss[b]).wait()
            pltpu.make_async_copy(ed_hbm.at[didx2.at[j]], edb[b], gds[b]).wait()
            for r in range(_CH):
                esb[b][r, :] = esb[b][r, :] + edb[b][r, :]
            pltpu.sync_copy(esb[b], out_hbm.at[pl.ds((w * _NCHUNK + j) * _CH, _CH)])

            @pl.when(j + _NB < _NCHUNK)
            def _():
                pltpu.async_copy(es_hbm.at[sidx2.at[j + _NB]], esb[b], gss[b])
                pltpu.async_copy(ed_hbm.at[didx2.at[j + _NB]], edb[b], gds[b])
        return carry

    lax.fori_loop(0, _NCHUNK // _NB, body, 0)


_edge_call = pl.kernel(
    _edge_body,
    out_type=jax.ShapeDtypeStruct((_E, 16), jnp.float32),
    mesh=_mesh,
    compiler_params=pltpu.CompilerParams(use_tc_tiling_on_sc=False),
    scratch_types=(
        [pltpu.VMEM((_NCHUNK, _CH), jnp.int32)] * 2
        + [pltpu.VMEM((_CH, 16), jnp.float32)] * 8
        + [pltpu.SemaphoreType.DMA] * 8
    ),
)


def _embed_body(ty_ref, emb_ref, wm0_ref, yinit_ref, m0_ref):
    ty = ty_ref[0, 0, :]
    oh = (ty[:, None] == lax.broadcasted_iota(jnp.int32, (_BN, _V), 1))
    oh = oh.astype(jnp.float32)
    yi = jnp.dot(oh, emb_ref[...], preferred_element_type=jnp.float32, precision=lax.Precision.HIGHEST)
    yinit_ref[...] = yi
    m0_ref[...] = jnp.dot(yi, wm0_ref[...], preferred_element_type=jnp.float32)


def _embed(ty3, emb, wm0):
    return pl.pallas_call(
        _embed_body,
        grid=(_GRID,),
        in_specs=[
            pl.BlockSpec((1, 1, _BN), lambda i: (i, 0, 0)),
            pl.BlockSpec((_V, _D), lambda i: (0, 0)),
            pl.BlockSpec((_D, _D), lambda i: (0, 0)),
        ],
        out_specs=[
            pl.BlockSpec((_BN, _D), lambda i: (i, 0)),
            pl.BlockSpec((_BN, _D), lambda i: (i, 0)),
        ],
        out_shape=[
            jax.ShapeDtypeStruct((_N, _D), jnp.float32),
            jax.ShapeDtypeStruct((_N, _D), jnp.float32),
        ],
    )(ty3, emb, wm0)


def _kv_body(x_ref, w_ref, o_ref):
    o_ref[0] = jnp.dot(x_ref[...], w_ref[0], preferred_element_type=jnp.float32)


def _kv(x, wkv):
    return pl.pallas_call(
        _kv_body,
        grid=(6,),
        in_specs=[
            pl.BlockSpec((_NX, _D), lambda i: (0, 0)),
            pl.BlockSpec((1, _D, _D), lambda i: (i, 0, 0)),
        ],
        out_specs=pl.BlockSpec((1, _NX, _D), lambda i: (i, 0, 0)),
        out_shape=jax.ShapeDtypeStruct((6, _NX, _D), jnp.float32),
    )(x, wkv)


def _block_body(agg_ref, y_ref, yi_ref, zc_ref, ws_ref, wi_ref, wet_ref,
                wq_ref, k_ref, v_ref, wp_ref, yo_ref, alpha_ref, post_ref):
    h = (agg_ref[0] + agg_ref[1]
         + jnp.dot(y_ref[...], ws_ref[...], preferred_element_type=jnp.float32)
         + jnp.dot(yi_ref[...], wi_ref[...], preferred_element_type=jnp.float32)
         + jnp.dot(zc_ref[...], wet_ref[...], preferred_element_type=jnp.float32,
                   precision=lax.Precision.HIGHEST))
    q = jnp.dot(h, wq_ref[...], preferred_element_type=jnp.float32)
    sc = lax.dot_general(q, k_ref[...], (((1,), (1,)), ((), ())),
                         preferred_element_type=jnp.float32)
    sc = sc * (1.0 / math.sqrt(_D))
    mx = jnp.max(sc, axis=1, keepdims=True)
    e = jnp.exp(sc - mx)
    a = e / jnp.sum(e, axis=1, keepdims=True)
    ctx = jnp.dot(a, v_ref[...], preferred_element_type=jnp.float32)
    yo = jnp.maximum(h + ctx, 0.0)
    yo_ref[...] = yo
    alpha_ref[...] = a
    post_ref[...] = jnp.dot(yo, wp_ref[...], preferred_element_type=jnp.float32)


def _block(aggp, y, y_init, zc, ws, wi, wet, wq, k, v, wpost):
    pd = wpost.shape[1]
    return pl.pallas_call(
        _block_body,
        grid=(_GRID,),
        in_specs=[
            pl.BlockSpec((_NC, _BN, _D), lambda i: (0, i, 0)),
            pl.BlockSpec((_BN, _D), lambda i: (i, 0)),
            pl.BlockSpec((_BN, _D), lambda i: (i, 0)),
            pl.BlockSpec((_BN, _D), lambda i: (i, 0)),
            pl.BlockSpec((_D, _D), lambda i: (0, 0)),
            pl.BlockSpec((_D, _D), lambda i: (0, 0)),
            pl.BlockSpec((_D, _D), lambda i: (0, 0)),
            pl.BlockSpec((_D, _D), lambda i: (0, 0)),
            pl.BlockSpec((_NX, _D), lambda i: (0, 0)),
            pl.BlockSpec((_NX, _D), lambda i: (0, 0)),
            pl.BlockSpec((_D, pd), lambda i: (0, 0)),
        ],
        out_specs=[
            pl.BlockSpec((_BN, _D), lambda i: (i, 0)),
            pl.BlockSpec((_BN, _NX), lambda i: (i, 0)),
            pl.BlockSpec((_BN, pd), lambda i: (i, 0)),
        ],
        out_shape=[
            jax.ShapeDtypeStruct((_N, _D), jnp.float32),
            jax.ShapeDtypeStruct((_N, _NX), jnp.float32),
            jax.ShapeDtypeStruct((_N, pd), jnp.float32),
        ],
    )(aggp, y, y_init, zc, ws, wi, wet, wq, k, v, wpost)


def kernel(x, x_batch, tgt_y, tgt_edge_index, tgt_edge_type, tgt_y_batch,
           emb, Wmsg, Wself, Winit, Et, Wq, Wk, Wv, Wnode, Wedge):
    f32 = jnp.float32
    src = tgt_edge_index[0]
    dst = tgt_edge_index[1]
    zeros16 = jnp.zeros((_NP, 16), f32)
    zerosD = jnp.zeros((_NP, _D), f32)

    src2 = src.reshape(_E // _CH, _CH)
    dst2 = dst.reshape(_E // _CH, _CH)
    et2 = tgt_edge_type.reshape(_E // _CH, _CH)
    cparts = _counts_call(dst2, et2, jnp.eye(16, dtype=f32), zeros16)
    counts16 = (cparts[0] + cparts[1])[:_N]

    ty3 = tgt_y.reshape(_GRID, 1, _BN)
    y_init, m = _embed(ty3, emb, Wmsg[0])

    kv = _kv(x, jnp.concatenate([Wk, Wv], axis=0))

    zc = jnp.concatenate([counts16, jnp.zeros((_N, 112), f32)], axis=1)

    wpost2 = jnp.concatenate([
        Wnode,
        Wedge[:_D], jnp.zeros((_D, 8), f32),
        Wedge[_D:], jnp.zeros((_D, 8), f32),
        jnp.zeros((_D, 96), f32),
    ], axis=1)

    alphas = []
    y = y_init
    for i in range(3):
        wet = jnp.concatenate([Et[i], jnp.zeros((124, _D), f32)], axis=0)
        aggp = _spmm_call(m, src2, dst2, zerosD)
        wpost = wpost2 if i == 2 else Wmsg[i + 1]
        y, a, post = _block(aggp, y, y_init, zc, Wself[i], Winit[i], wet,
                            Wq[i], kv[i], kv[3 + i], wpost)
        alphas.append(a)
        m = post if i < 2 else None

    y_score = post[:, :_V]
    es16 = post[:, _V:_V + 16]
    ed16 = post[:, _V + 16:_V + 32]
    edge16 = _edge_call(es16, ed16, src2, dst2)
    y_edge_rel_score = edge16[:, :8]

    return (y_score, y_edge_rel_score, alphas[0], alphas[1], alphas[2])


# speedup vs baseline: 10.6952x; 2.9585x over previous
import functools
import math

import jax
import jax.numpy as jnp
from jax import lax
from jax.experimental import pallas as pl
from jax.experimental.pallas import tpu as pltpu
from jax.experimental.pallas import tpu_sc as plsc

_N = 10000
_E = 320000
_NX = 1024
_D = 128
_V = 512
_NC = 2
_NS = 16
_NW = _NC * _NS
_EPW = _E // _NW
_CH = 125
_NCHUNK = _EPW // _CH
_NB = 4
_NP = 10112
_RPW = _NP // _NS
_BN = 400
_GRID = _N // _BN

_mesh = plsc.VectorSubcoreMesh(core_axis_name="c", subcore_axis_name="s")


def _wid():
    return lax.axis_index("s") * _NC + lax.axis_index("c")


def _counts_body(dst2_hbm, et2_hbm, zeros16_hbm, out_hbm,
                 didx2, etv2, oh0, oh1, oh2, oh3, cnt_sh, g0, g1, g2, g3):
    c = lax.axis_index("c")
    s = lax.axis_index("s")
    w = _wid()
    pltpu.sync_copy(zeros16_hbm.at[pl.ds(s * _RPW, _RPW)],
                    cnt_sh.at[pl.ds(s * _RPW, _RPW)])
    pltpu.sync_copy(dst2_hbm.at[pl.ds(w * _NCHUNK, _NCHUNK)], didx2)
    pltpu.sync_copy(et2_hbm.at[pl.ds(w * _NCHUNK, _NCHUNK)], etv2)
    plsc.subcore_barrier()
    bufs = (oh0, oh1, oh2, oh3)
    sems = (g0, g1, g2, g3)
    lanes = lax.iota(jnp.int32, 16)

    def body(i, carry):
        for b in range(4):
            j = i * 4 + b

            @pl.when(j >= 4)
            def _():
                pltpu.make_async_copy(bufs[b], cnt_sh.at[didx2.at[j]],
                                      sems[b]).wait()
            for g in range(8):
                base_r = min(g * 16, _CH - 16)
                etv = etv2[j, pl.ds(base_r, 16)]
                for r in range(16):
                    bufs[b][base_r + r, :] = jnp.where(lanes == etv[r], 1.0, 0.0)
            pltpu.async_copy(bufs[b], cnt_sh.at[didx2.at[j]], sems[b])
        return carry

    lax.fori_loop(0, _NCHUNK // 4, body, 0)
    for b in range(4):
        pltpu.make_async_copy(bufs[b], cnt_sh.at[didx2.at[_NCHUNK - 4 + b]],
                              sems[b]).wait()
    plsc.subcore_barrier()
    pltpu.sync_copy(cnt_sh.at[pl.ds(s * _RPW, _RPW)],
                    out_hbm.at[c, pl.ds(s * _RPW, _RPW)])


_counts_call = pl.kernel(
    _counts_body,
    out_type=jax.ShapeDtypeStruct((_NC, _NP, 16), jnp.float32),
    mesh=_mesh,
    compiler_params=pltpu.CompilerParams(use_tc_tiling_on_sc=False),
    scratch_types=(
        [pltpu.VMEM((_NCHUNK, _CH), jnp.int32)] * 2
        + [pltpu.VMEM((_CH, 16), jnp.float32)] * 4
        + [pltpu.VMEM_SHARED((_NP, 16), jnp.float32)]
        + [pltpu.SemaphoreType.DMA] * 4
    ),
)


def _spmm_body(m_hbm, src2_hbm, dst2_hbm, zeros_hbm, out_hbm,
               si0, si1, si2, si3, di0, di1, di2, di3, r0, r1, agg_sh,
               is0, is1, is2, is3, id0, id1, id2, id3, g0, g1):
    c = lax.axis_index("c")
    s = lax.axis_index("s")
    w = _wid()
    pltpu.sync_copy(zeros_hbm.at[pl.ds(s * _RPW, _RPW)],
                    agg_sh.at[pl.ds(s * _RPW, _RPW)])
    plsc.subcore_barrier()
    sib = (si0, si1, si2, si3)
    dib = (di0, di1, di2, di3)
    isem = (is0, is1, is2, is3)
    idsem = (id0, id1, id2, id3)
    rows = (r0, r1)
    gsem = (g0, g1)
    cb = w * _NCHUNK

    def idx_start(j, sl):
        pltpu.async_copy(src2_hbm.at[cb + j], sib[sl], isem[sl])
        pltpu.async_copy(dst2_hbm.at[cb + j], dib[sl], idsem[sl])

    def idx_wait(j, sl):
        pltpu.make_async_copy(src2_hbm.at[cb + j], sib[sl], isem[sl]).wait()
        pltpu.make_async_copy(dst2_hbm.at[cb + j], dib[sl], idsem[sl]).wait()

    for sl in range(4):
        idx_start(sl, sl)
    idx_wait(0, 0)
    idx_wait(1, 1)
    pltpu.async_copy(m_hbm.at[sib[0]], rows[0], gsem[0])
    pltpu.async_copy(m_hbm.at[sib[1]], rows[1], gsem[1])

    def body(i, carry):
        for b4 in range(4):
            j = i * 4 + b4
            b2 = b4 % 2

            pltpu.make_async_copy(m_hbm.at[sib[b4]], rows[b2], gsem[b2]).wait()
            pltpu.sync_copy(rows[b2], agg_sh.at[dib[b4]], add=True)

            @pl.when(j + 4 < _NCHUNK)
            def _():
                idx_start(j + 4, b4)

            @pl.when(j + 2 < _NCHUNK)
            def _():
                sl = (b4 + 2) % 4
                idx_wait(j + 2, sl)
                pltpu.async_copy(m_hbm.at[sib[sl]], rows[b2], gsem[b2])
        return carry

    lax.fori_loop(0, _NCHUNK // 4, body, 0)
    plsc.subcore_barrier()
    pltpu.sync_copy(agg_sh.at[pl.ds(s * _RPW, _RPW)],
                    out_hbm.at[c, pl.ds(s * _RPW, _RPW)])


_spmm_call = pl.kernel(
    _spmm_body,
    out_type=jax.ShapeDtypeStruct((_NC, _NP, _D), jnp.float32),
    mesh=_mesh,
    scratch_types=(
        [pltpu.VMEM((_CH,), jnp.int32)] * 8
        + [pltpu.VMEM((_CH, _D), jnp.float32)] * 2
        + [pltpu.VMEM_SHARED((_NP, _D), jnp.float32)]
        + [pltpu.SemaphoreType.DMA] * 10
    ),
)


def _edge_body(es_hbm, ed_hbm, src2_hbm, dst2_hbm, out_hbm,
               sidx2, didx2, es0, es1, es2, es3, ed0, ed1, ed2, ed3,
               gs0, gs1, gs2, gs3, gd0, gd1, gd2, gd3):
    w = _wid()
    pltpu.sync_copy(src2_hbm.at[pl.ds(w * _NCHUNK, _NCHUNK)], sidx2)
    pltpu.sync_copy(dst2_hbm.at[pl.ds(w * _NCHUNK, _NCHUNK)], didx2)
    esb = (es0, es1, es2, es3)
    edb = (ed0, ed1, ed2, ed3)
    gss = (gs0, gs1, gs2, gs3)
    gds = (gd0, gd1, gd2, gd3)
    for b in range(_NB):
        pltpu.async_copy(es_hbm.at[sidx2.at[b]], esb[b], gss[b])
        pltpu.async_copy(ed_hbm.at[didx2.at[b]], edb[b], gds[b])

    def body(i, carry):
        for b in range(_NB):
            j = i * _NB + b
            pltpu.make_async_copy(es_hbm.at[sidx2.at[j]], esb[b], gss[b]).wait()
            pltpu.make_async_copy(ed_hbm.at[didx2.at[j]], edb[b], gds[b]).wait()
            for r in range(_CH):
                esb[b][r, :] = esb[b][r, :] + edb[b][r, :]
            pltpu.sync_copy(esb[b], out_hbm.at[pl.ds((w * _NCHUNK + j) * _CH, _CH)])

            @pl.when(j + _NB < _NCHUNK)
            def _():
                pltpu.async_copy(es_hbm.at[sidx2.at[j + _NB]], esb[b], gss[b])
                pltpu.async_copy(ed_hbm.at[didx2.at[j + _NB]], edb[b], gds[b])
        return carry

    lax.fori_loop(0, _NCHUNK // _NB, body, 0)


_edge_call = pl.kernel(
    _edge_body,
    out_type=jax.ShapeDtypeStruct((_E, 16), jnp.float32),
    mesh=_mesh,
    compiler_params=pltpu.CompilerParams(use_tc_tiling_on_sc=False),
    scratch_types=(
        [pltpu.VMEM((_NCHUNK, _CH), jnp.int32)] * 2
        + [pltpu.VMEM((_CH, 16), jnp.float32)] * 8
        + [pltpu.SemaphoreType.DMA] * 8
    ),
)


def _embed_body(ty_ref, emb_ref, wm0_ref, yinit_ref, m0_ref):
    ty = ty_ref[0, 0, :]
    oh = (ty[:, None] == lax.broadcasted_iota(jnp.int32, (_BN, _V), 1))
    oh = oh.astype(jnp.float32)
    yi = jnp.dot(oh, emb_ref[...], preferred_element_type=jnp.float32, precision=lax.Precision.HIGHEST)
    yinit_ref[...] = yi
    m0_ref[...] = jnp.dot(yi, wm0_ref[...], preferred_element_type=jnp.float32)


def _embed(ty3, emb, wm0):
    return pl.pallas_call(
        _embed_body,
        grid=(_GRID,),
        in_specs=[
            pl.BlockSpec((1, 1, _BN), lambda i: (i, 0, 0)),
            pl.BlockSpec((_V, _D), lambda i: (0, 0)),
            pl.BlockSpec((_D, _D), lambda i: (0, 0)),
        ],
        out_specs=[
            pl.BlockSpec((_BN, _D), lambda i: (i, 0)),
            pl.BlockSpec((_BN, _D), lambda i: (i, 0)),
        ],
        out_shape=[
            jax.ShapeDtypeStruct((_N, _D), jnp.float32),
            jax.ShapeDtypeStruct((_N, _D), jnp.float32),
        ],
    )(ty3, emb, wm0)


def _kv_body(x_ref, w_ref, o_ref):
    o_ref[0] = jnp.dot(x_ref[...], w_ref[0], preferred_element_type=jnp.float32)


def _kv(x, wkv):
    return pl.pallas_call(
        _kv_body,
        grid=(6,),
        in_specs=[
            pl.BlockSpec((_NX, _D), lambda i: (0, 0)),
            pl.BlockSpec((1, _D, _D), lambda i: (i, 0, 0)),
        ],
        out_specs=pl.BlockSpec((1, _NX, _D), lambda i: (i, 0, 0)),
        out_shape=jax.ShapeDtypeStruct((6, _NX, _D), jnp.float32),
    )(x, wkv)


def _block_body(agg_ref, y_ref, yi_ref, zc_ref, ws_ref, wi_ref, wet_ref,
                wq_ref, k_ref, v_ref, wp_ref, yo_ref, alpha_ref, post_ref):
    h = (agg_ref[0] + agg_ref[1]
         + jnp.dot(y_ref[...], ws_ref[...], preferred_element_type=jnp.float32)
         + jnp.dot(yi_ref[...], wi_ref[...], preferred_element_type=jnp.float32)
         + jnp.dot(zc_ref[...], wet_ref[...], preferred_element_type=jnp.float32,
                   precision=lax.Precision.HIGHEST))
    q = jnp.dot(h, wq_ref[...], preferred_element_type=jnp.float32)
    sc = lax.dot_general(q, k_ref[...], (((1,), (1,)), ((), ())),
                         preferred_element_type=jnp.float32)
    sc = sc * (1.0 / math.sqrt(_D))
    mx = jnp.max(sc, axis=1, keepdims=True)
    e = jnp.exp(sc - mx)
    a = e / jnp.sum(e, axis=1, keepdims=True)
    ctx = jnp.dot(a, v_ref[...], preferred_element_type=jnp.float32)
    yo = jnp.maximum(h + ctx, 0.0)
    yo_ref[...] = yo
    alpha_ref[...] = a
    post_ref[...] = jnp.dot(yo, wp_ref[...], preferred_element_type=jnp.float32)


def _block(aggp, y, y_init, zc, ws, wi, wet, wq, k, v, wpost):
    pd = wpost.shape[1]
    return pl.pallas_call(
        _block_body,
        grid=(_GRID,),
        in_specs=[
            pl.BlockSpec((_NC, _BN, _D), lambda i: (0, i, 0)),
            pl.BlockSpec((_BN, _D), lambda i: (i, 0)),
            pl.BlockSpec((_BN, _D), lambda i: (i, 0)),
            pl.BlockSpec((_BN, _D), lambda i: (i, 0)),
            pl.BlockSpec((_D, _D), lambda i: (0, 0)),
            pl.BlockSpec((_D, _D), lambda i: (0, 0)),
            pl.BlockSpec((_D, _D), lambda i: (0, 0)),
            pl.BlockSpec((_D, _D), lambda i: (0, 0)),
            pl.BlockSpec((_NX, _D), lambda i: (0, 0)),
            pl.BlockSpec((_NX, _D), lambda i: (0, 0)),
            pl.BlockSpec((_D, pd), lambda i: (0, 0)),
        ],
        out_specs=[
            pl.BlockSpec((_BN, _D), lambda i: (i, 0)),
            pl.BlockSpec((_BN, _NX), lambda i: (i, 0)),
            pl.BlockSpec((_BN, pd), lambda i: (i, 0)),
        ],
        out_shape=[
            jax.ShapeDtypeStruct((_N, _D), jnp.float32),
            jax.ShapeDtypeStruct((_N, _NX), jnp.float32),
            jax.ShapeDtypeStruct((_N, pd), jnp.float32),
        ],
    )(aggp, y, y_init, zc, ws, wi, wet, wq, k, v, wpost)


def kernel(x, x_batch, tgt_y, tgt_edge_index, tgt_edge_type, tgt_y_batch,
           emb, Wmsg, Wself, Winit, Et, Wq, Wk, Wv, Wnode, Wedge):
    f32 = jnp.float32
    src = tgt_edge_index[0]
    dst = tgt_edge_index[1]
    zeros16 = jnp.zeros((_NP, 16), f32)
    zerosD = jnp.zeros((_NP, _D), f32)

    src2 = src.reshape(_E // _CH, _CH)
    dst2 = dst.reshape(_E // _CH, _CH)
    et2 = tgt_edge_type.reshape(_E // _CH, _CH)
    cparts = _counts_call(dst2, et2, zeros16)
    counts16 = (cparts[0] + cparts[1])[:_N]

    ty3 = tgt_y.reshape(_GRID, 1, _BN)
    y_init, m = _embed(ty3, emb, Wmsg[0])

    kv = _kv(x, jnp.concatenate([Wk, Wv], axis=0))

    zc = jnp.concatenate([counts16, jnp.zeros((_N, 112), f32)], axis=1)

    wpost2 = jnp.concatenate([
        Wnode,
        Wedge[:_D], jnp.zeros((_D, 8), f32),
        Wedge[_D:], jnp.zeros((_D, 8), f32),
        jnp.zeros((_D, 96), f32),
    ], axis=1)

    alphas = []
    y = y_init
    for i in range(3):
        wet = jnp.concatenate([Et[i], jnp.zeros((124, _D), f32)], axis=0)
        aggp = _spmm_call(m, src2, dst2, zerosD)
        wpost = wpost2 if i == 2 else Wmsg[i + 1]
        y, a, post = _block(aggp, y, y_init, zc, Wself[i], Winit[i], wet,
                            Wq[i], kv[i], kv[3 + i], wpost)
        alphas.append(a)
        m = post if i < 2 else None

    y_score = post[:, :_V]
    es16 = post[:, _V:_V + 16]
    ed16 = post[:, _V + 16:_V + 32]
    edge16 = _edge_call(es16, ed16, src2, dst2)
    y_edge_rel_score = edge16[:, :8]

    return (y_score, y_edge_rel_score, alphas[0], alphas[1], alphas[2])


# speedup vs baseline: 10.7080x; 1.0012x over previous
import functools
import math

import jax
import jax.numpy as jnp
from jax import lax
from jax.experimental import pallas as pl
from jax.experimental.pallas import tpu as pltpu
from jax.experimental.pallas import tpu_sc as plsc

_N = 10000
_E = 320000
_NX = 1024
_D = 128
_V = 512
_NC = 2
_NS = 16
_NW = _NC * _NS
_EPW = _E // _NW
_CH = 125
_NCHUNK = _EPW // _CH
_NB = 4
_NP = 10112
_RPW = _NP // _NS
_BN = 400
_GRID = _N // _BN

_mesh = plsc.VectorSubcoreMesh(core_axis_name="c", subcore_axis_name="s")


def _wid():
    return lax.axis_index("s") * _NC + lax.axis_index("c")


def _counts_body(dst2_hbm, et2_hbm, zeros16_hbm, out_hbm,
                 didx2, etv2, oh0, oh1, oh2, oh3, cnt_sh, g0, g1, g2, g3):
    c = lax.axis_index("c")
    s = lax.axis_index("s")
    w = _wid()
    pltpu.sync_copy(zeros16_hbm.at[pl.ds(s * _RPW, _RPW)],
                    cnt_sh.at[pl.ds(s * _RPW, _RPW)])
    pltpu.sync_copy(dst2_hbm.at[pl.ds(w * _NCHUNK, _NCHUNK)], didx2)
    pltpu.sync_copy(et2_hbm.at[pl.ds(w * _NCHUNK, _NCHUNK)], etv2)
    plsc.subcore_barrier()
    bufs = (oh0, oh1, oh2, oh3)
    sems = (g0, g1, g2, g3)
    lanes = lax.iota(jnp.int32, 16)

    def body(i, carry):
        for b in range(4):
            j = i * 4 + b

            for g in range(8):
                base_r = min(g * 16, _CH - 16)
                etv = etv2[j, pl.ds(base_r, 16)]
                for r in range(16):
                    bufs[b][base_r + r, :] = jnp.where(lanes == etv[r], 1.0, 0.0)
            pltpu.sync_copy(bufs[b], cnt_sh.at[didx2.at[j]], add=True)
        return carry

    lax.fori_loop(0, _NCHUNK // 4, body, 0)
    plsc.subcore_barrier()
    pltpu.sync_copy(cnt_sh.at[pl.ds(s * _RPW, _RPW)],
                    out_hbm.at[c, pl.ds(s * _RPW, _RPW)])


_counts_call = pl.kernel(
    _counts_body,
    out_type=jax.ShapeDtypeStruct((_NC, _NP, 16), jnp.float32),
    mesh=_mesh,
    compiler_params=pltpu.CompilerParams(use_tc_tiling_on_sc=False),
    scratch_types=(
        [pltpu.VMEM((_NCHUNK, _CH), jnp.int32)] * 2
        + [pltpu.VMEM((_CH, 16), jnp.float32)] * 4
        + [pltpu.VMEM_SHARED((_NP, 16), jnp.float32)]
        + [pltpu.SemaphoreType.DMA] * 4
    ),
)


def _spmm_body(m_hbm, src2_hbm, dst2_hbm, zeros_hbm, out_hbm,
               si0, si1, si2, si3, di0, di1, di2, di3, r0, r1, agg_sh,
               is0, is1, is2, is3, id0, id1, id2, id3, g0, g1):
    c = lax.axis_index("c")
    s = lax.axis_index("s")
    w = _wid()
    pltpu.sync_copy(zeros_hbm.at[pl.ds(s * _RPW, _RPW)],
                    agg_sh.at[pl.ds(s * _RPW, _RPW)])
    plsc.subcore_barrier()
    sib = (si0, si1, si2, si3)
    dib = (di0, di1, di2, di3)
    isem = (is0, is1, is2, is3)
    idsem = (id0, id1, id2, id3)
    rows = (r0, r1)
    gsem = (g0, g1)
    cb = w * _NCHUNK

    def idx_start(j, sl):
        pltpu.async_copy(src2_hbm.at[cb + j], sib[sl], isem[sl])
        pltpu.async_copy(dst2_hbm.at[cb + j], dib[sl], idsem[sl])

    def idx_wait(j, sl):
        pltpu.make_async_copy(src2_hbm.at[cb + j], sib[sl], isem[sl]).wait()
        pltpu.make_async_copy(dst2_hbm.at[cb + j], dib[sl], idsem[sl]).wait()

    for sl in range(4):
        idx_start(sl, sl)
    idx_wait(0, 0)
    idx_wait(1, 1)
    pltpu.async_copy(m_hbm.at[sib[0]], rows[0], gsem[0])
    pltpu.async_copy(m_hbm.at[sib[1]], rows[1], gsem[1])

    def body(i, carry):
        for b4 in range(4):
            j = i * 4 + b4
            b2 = b4 % 2

            pltpu.make_async_copy(m_hbm.at[sib[b4]], rows[b2], gsem[b2]).wait()
            pltpu.sync_copy(rows[b2], agg_sh.at[dib[b4]], add=True)

            @pl.when(j + 4 < _NCHUNK)
            def _():
                idx_start(j + 4, b4)

            @pl.when(j + 2 < _NCHUNK)
            def _():
                sl = (b4 + 2) % 4
                idx_wait(j + 2, sl)
                pltpu.async_copy(m_hbm.at[sib[sl]], rows[b2], gsem[b2])
        return carry

    lax.fori_loop(0, _NCHUNK // 4, body, 0)
    plsc.subcore_barrier()
    pltpu.sync_copy(agg_sh.at[pl.ds(s * _RPW, _RPW)],
                    out_hbm.at[c, pl.ds(s * _RPW, _RPW)])


_spmm_call = pl.kernel(
    _spmm_body,
    out_type=jax.ShapeDtypeStruct((_NC, _NP, _D), jnp.float32),
    mesh=_mesh,
    scratch_types=(
        [pltpu.VMEM((_CH,), jnp.int32)] * 8
        + [pltpu.VMEM((_CH, _D), jnp.float32)] * 2
        + [pltpu.VMEM_SHARED((_NP, _D), jnp.float32)]
        + [pltpu.SemaphoreType.DMA] * 10
    ),
)


def _edge_body(es_hbm, ed_hbm, src2_hbm, dst2_hbm, out_hbm,
               sidx2, didx2, es0, es1, es2, es3, ed0, ed1, ed2, ed3,
               gs0, gs1, gs2, gs3, gd0, gd1, gd2, gd3):
    w = _wid()
    pltpu.sync_copy(src2_hbm.at[pl.ds(w * _NCHUNK, _NCHUNK)], sidx2)
    pltpu.sync_copy(dst2_hbm.at[pl.ds(w * _NCHUNK, _NCHUNK)], didx2)
    esb = (es0, es1, es2, es3)
    edb = (ed0, ed1, ed2, ed3)
    gss = (gs0, gs1, gs2, gs3)
    gds = (gd0, gd1, gd2, gd3)
    for b in range(_NB):
        pltpu.async_copy(es_hbm.at[sidx2.at[b]], esb[b], gss[b])
        pltpu.async_copy(ed_hbm.at[didx2.at[b]], edb[b], gds[b])

    def body(i, carry):
        for b in range(_NB):
            j = i * _NB + b
            pltpu.make_async_copy(es_hbm.at[sidx2.at[j]], esb[b], gss[b]).wait()
            pltpu.make_async_copy(ed_hbm.at[didx2.at[j]], edb[b], gds[b]).wait()
            for r in range(_CH):
                esb[b][r, :] = esb[b][r, :] + edb[b][r, :]
            pltpu.sync_copy(esb[b], out_hbm.at[pl.ds((w * _NCHUNK + j) * _CH, _CH)])

            @pl.when(j + _NB < _NCHUNK)
            def _():
                pltpu.async_copy(es_hbm.at[sidx2.at[j + _NB]], esb[b], gss[b])
                pltpu.async_copy(ed_hbm.at[didx2.at[j + _NB]], edb[b], gds[b])
        return carry

    lax.fori_loop(0, _NCHUNK // _NB, body, 0)


_edge_call = pl.kernel(
    _edge_body,
    out_type=jax.ShapeDtypeStruct((_E, 16), jnp.float32),
    mesh=_mesh,
    compiler_params=pltpu.CompilerParams(use_tc_tiling_on_sc=False),
    scratch_types=(
        [pltpu.VMEM((_NCHUNK, _CH), jnp.int32)] * 2
        + [pltpu.VMEM((_CH, 16), jnp.float32)] * 8
        + [pltpu.SemaphoreType.DMA] * 8
    ),
)


def _embed_body(ty_ref, emb_ref, wm0_ref, yinit_ref, m0_ref):
    ty = ty_ref[0, 0, :]
    oh = (ty[:, None] == lax.broadcasted_iota(jnp.int32, (_BN, _V), 1))
    oh = oh.astype(jnp.float32)
    yi = jnp.dot(oh, emb_ref[...], preferred_element_type=jnp.float32, precision=lax.Precision.HIGHEST)
    yinit_ref[...] = yi
    m0_ref[...] = jnp.dot(yi, wm0_ref[...], preferred_element_type=jnp.float32)


def _embed(ty3, emb, wm0):
    return pl.pallas_call(
        _embed_body,
        grid=(_GRID,),
        in_specs=[
            pl.BlockSpec((1, 1, _BN), lambda i: (i, 0, 0)),
            pl.BlockSpec((_V, _D), lambda i: (0, 0)),
            pl.BlockSpec((_D, _D), lambda i: (0, 0)),
        ],
        out_specs=[
            pl.BlockSpec((_BN, _D), lambda i: (i, 0)),
            pl.BlockSpec((_BN, _D), lambda i: (i, 0)),
        ],
        out_shape=[
            jax.ShapeDtypeStruct((_N, _D), jnp.float32),
            jax.ShapeDtypeStruct((_N, _D), jnp.float32),
        ],
    )(ty3, emb, wm0)


def _kv_body(x_ref, w_ref, o_ref):
    o_ref[0] = jnp.dot(x_ref[...], w_ref[0], preferred_element_type=jnp.float32)


def _kv(x, wkv):
    return pl.pallas_call(
        _kv_body,
        grid=(6,),
        in_specs=[
            pl.BlockSpec((_NX, _D), lambda i: (0, 0)),
            pl.BlockSpec((1, _D, _D), lambda i: (i, 0, 0)),
        ],
        out_specs=pl.BlockSpec((1, _NX, _D), lambda i: (i, 0, 0)),
        out_shape=jax.ShapeDtypeStruct((6, _NX, _D), jnp.float32),
    )(x, wkv)


def _block_body(agg_ref, y_ref, yi_ref, zc_ref, ws_ref, wi_ref, wet_ref,
                wq_ref, k_ref, v_ref, wp_ref, yo_ref, alpha_ref, post_ref):
    h = (agg_ref[0] + agg_ref[1]
         + jnp.dot(y_ref[...], ws_ref[...], preferred_element_type=jnp.float32)
         + jnp.dot(yi_ref[...], wi_ref[...], preferred_element_type=jnp.float32)
         + jnp.dot(zc_ref[...], wet_ref[...], preferred_element_type=jnp.float32,
                   precision=lax.Precision.HIGHEST))
    q = jnp.dot(h, wq_ref[...], preferred_element_type=jnp.float32)
    sc = lax.dot_general(q, k_ref[...], (((1,), (1,)), ((), ())),
                         preferred_element_type=jnp.float32)
    sc = sc * (1.0 / math.sqrt(_D))
    mx = jnp.max(sc, axis=1, keepdims=True)
    e = jnp.exp(sc - mx)
    a = e / jnp.sum(e, axis=1, keepdims=True)
    ctx = jnp.dot(a, v_ref[...], preferred_element_type=jnp.float32)
    yo = jnp.maximum(h + ctx, 0.0)
    yo_ref[...] = yo
    alpha_ref[...] = a
    post_ref[...] = jnp.dot(yo, wp_ref[...], preferred_element_type=jnp.float32)


def _block(aggp, y, y_init, zc, ws, wi, wet, wq, k, v, wpost):
    pd = wpost.shape[1]
    return pl.pallas_call(
        _block_body,
        grid=(_GRID,),
        in_specs=[
            pl.BlockSpec((_NC, _BN, _D), lambda i: (0, i, 0)),
            pl.BlockSpec((_BN, _D), lambda i: (i, 0)),
            pl.BlockSpec((_BN, _D), lambda i: (i, 0)),
            pl.BlockSpec((_BN, _D), lambda i: (i, 0)),
            pl.BlockSpec((_D, _D), lambda i: (0, 0)),
            pl.BlockSpec((_D, _D), lambda i: (0, 0)),
            pl.BlockSpec((_D, _D), lambda i: (0, 0)),
            pl.BlockSpec((_D, _D), lambda i: (0, 0)),
            pl.BlockSpec((_NX, _D), lambda i: (0, 0)),
            pl.BlockSpec((_NX, _D), lambda i: (0, 0)),
            pl.BlockSpec((_D, pd), lambda i: (0, 0)),
        ],
        out_specs=[
            pl.BlockSpec((_BN, _D), lambda i: (i, 0)),
            pl.BlockSpec((_BN, _NX), lambda i: (i, 0)),
            pl.BlockSpec((_BN, pd), lambda i: (i, 0)),
        ],
        out_shape=[
            jax.ShapeDtypeStruct((_N, _D), jnp.float32),
            jax.ShapeDtypeStruct((_N, _NX), jnp.float32),
            jax.ShapeDtypeStruct((_N, pd), jnp.float32),
        ],
    )(aggp, y, y_init, zc, ws, wi, wet, wq, k, v, wpost)


def kernel(x, x_batch, tgt_y, tgt_edge_index, tgt_edge_type, tgt_y_batch,
           emb, Wmsg, Wself, Winit, Et, Wq, Wk, Wv, Wnode, Wedge):
    f32 = jnp.float32
    src = tgt_edge_index[0]
    dst = tgt_edge_index[1]
    zeros16 = jnp.zeros((_NP, 16), f32)
    zerosD = jnp.zeros((_NP, _D), f32)

    src2 = src.reshape(_E // _CH, _CH)
    dst2 = dst.reshape(_E // _CH, _CH)
    et2 = tgt_edge_type.reshape(_E // _CH, _CH)
    cparts = _counts_call(dst2, et2, zeros16)
    counts16 = (cparts[0] + cparts[1])[:_N]

    ty3 = tgt_y.reshape(_GRID, 1, _BN)
    y_init, m = _embed(ty3, emb, Wmsg[0])

    kv = _kv(x, jnp.concatenate([Wk, Wv], axis=0))

    zc = jnp.concatenate([counts16, jnp.zeros((_N, 112), f32)], axis=1)

    wpost2 = jnp.concatenate([
        Wnode,
        Wedge[:_D], jnp.zeros((_D, 8), f32),
        Wedge[_D:], jnp.zeros((_D, 8), f32),
        jnp.zeros((_D, 96), f32),
    ], axis=1)

    alphas = []
    y = y_init
    for i in range(3):
        wet = jnp.concatenate([Et[i], jnp.zeros((124, _D), f32)], axis=0)
        aggp = _spmm_call(m, src2, dst2, zerosD)
        wpost = wpost2 if i == 2 else Wmsg[i + 1]
        y, a, post = _block(aggp, y, y_init, zc, Wself[i], Winit[i], wet,
                            Wq[i], kv[i], kv[3 + i], wpost)
        alphas.append(a)
        m = post if i < 2 else None

    y_score = post[:, :_V]
    es16 = post[:, _V:_V + 16]
    ed16 = post[:, _V + 16:_V + 32]
    edge16 = _edge_call(es16, ed16, src2, dst2)
    y_edge_rel_score = edge16[:, :8]

    return (y_score, y_edge_rel_score, alphas[0], alphas[1], alphas[2])


# speedup vs baseline: 10.7137x; 1.0005x over previous
import functools
import math

import jax
import jax.numpy as jnp
from jax import lax
from jax.experimental import pallas as pl
from jax.experimental.pallas import tpu as pltpu
from jax.experimental.pallas import tpu_sc as plsc

_N = 10000
_E = 320000
_NX = 1024
_D = 128
_V = 512
_NC = 2
_NS = 16
_NW = _NC * _NS
_EPW = _E // _NW
_CH = 125
_NCHUNK = _EPW // _CH
_NB = 4
_NP = 10112
_RPW = _NP // _NS
_BN = 400
_GRID = _N // _BN

_mesh = plsc.VectorSubcoreMesh(core_axis_name="c", subcore_axis_name="s")


def _wid():
    return lax.axis_index("s") * _NC + lax.axis_index("c")


def _counts_body(dst2_hbm, et2_hbm, zeros16_hbm, out_hbm,
                 didx2, etv2, oh0, oh1, oh2, oh3, cnt_sh, g0, g1, g2, g3):
    c = lax.axis_index("c")
    s = lax.axis_index("s")
    w = _wid()
    pltpu.sync_copy(zeros16_hbm.at[pl.ds(s * _RPW, _RPW)],
                    cnt_sh.at[pl.ds(s * _RPW, _RPW)])
    pltpu.sync_copy(dst2_hbm.at[pl.ds(w * _NCHUNK, _NCHUNK)], didx2)
    pltpu.sync_copy(et2_hbm.at[pl.ds(w * _NCHUNK, _NCHUNK)], etv2)
    plsc.subcore_barrier()
    bufs = (oh0, oh1, oh2, oh3)
    sems = (g0, g1, g2, g3)
    lanes = lax.iota(jnp.int32, 16)

    def body(i, carry):
        for b in range(4):
            j = i * 4 + b

            for g in range(8):
                base_r = min(g * 16, _CH - 16)
                etv = etv2[j, pl.ds(base_r, 16)]
                for r in range(16):
                    bufs[b][base_r + r, :] = jnp.where(lanes == etv[r], 1.0, 0.0)
            pltpu.sync_copy(bufs[b], cnt_sh.at[didx2.at[j]], add=True)
        return carry

    lax.fori_loop(0, _NCHUNK // 4, body, 0)
    plsc.subcore_barrier()
    pltpu.sync_copy(cnt_sh.at[pl.ds(s * _RPW, _RPW)],
                    out_hbm.at[c, pl.ds(s * _RPW, _RPW)])


_counts_call = pl.kernel(
    _counts_body,
    out_type=jax.ShapeDtypeStruct((_NC, _NP, 16), jnp.float32),
    mesh=_mesh,
    compiler_params=pltpu.CompilerParams(use_tc_tiling_on_sc=False),
    scratch_types=(
        [pltpu.VMEM((_NCHUNK, _CH), jnp.int32)] * 2
        + [pltpu.VMEM((_CH, 16), jnp.float32)] * 4
        + [pltpu.VMEM_SHARED((_NP, 16), jnp.float32)]
        + [pltpu.SemaphoreType.DMA] * 4
    ),
)


def _spmm_body(m_hbm, src2_hbm, dst2_hbm, zeros_hbm, out_hbm,
               si0, si1, si2, si3, di0, di1, di2, di3, r0, r1, agg_sh,
               is0, is1, is2, is3, id0, id1, id2, id3, g0, g1):
    c = lax.axis_index("c")
    s = lax.axis_index("s")
    w = _wid()
    pltpu.sync_copy(zeros_hbm.at[pl.ds(s * _RPW, _RPW)],
                    agg_sh.at[pl.ds(s * _RPW, _RPW)])
    plsc.subcore_barrier()
    sib = (si0, si1, si2, si3)
    dib = (di0, di1, di2, di3)
    isem = (is0, is1, is2, is3)
    idsem = (id0, id1, id2, id3)
    rows = (r0, r1)
    gsem = (g0, g1)
    cb = w * _NCHUNK

    def idx_start(j, sl):
        pltpu.async_copy(src2_hbm.at[cb + j], sib[sl], isem[sl])
        pltpu.async_copy(dst2_hbm.at[cb + j], dib[sl], idsem[sl])

    def idx_wait(j, sl):
        pltpu.make_async_copy(src2_hbm.at[cb + j], sib[sl], isem[sl]).wait()
        pltpu.make_async_copy(dst2_hbm.at[cb + j], dib[sl], idsem[sl]).wait()

    for sl in range(4):
        idx_start(sl, sl)
    idx_wait(0, 0)
    idx_wait(1, 1)
    pltpu.async_copy(m_hbm.at[sib[0]], rows[0], gsem[0])
    pltpu.async_copy(m_hbm.at[sib[1]], rows[1], gsem[1])

    def body(i, carry):
        for b4 in range(4):
            j = i * 4 + b4
            b2 = b4 % 2

            pltpu.make_async_copy(m_hbm.at[sib[b4]], rows[b2], gsem[b2]).wait()
            pltpu.sync_copy(rows[b2], agg_sh.at[dib[b4]], add=True)

            @pl.when(j + 4 < _NCHUNK)
            def _():
                idx_start(j + 4, b4)

            @pl.when(j + 2 < _NCHUNK)
            def _():
                sl = (b4 + 2) % 4
                idx_wait(j + 2, sl)
                pltpu.async_copy(m_hbm.at[sib[sl]], rows[b2], gsem[b2])
        return carry

    lax.fori_loop(0, _NCHUNK // 4, body, 0)
    plsc.subcore_barrier()
    pltpu.sync_copy(agg_sh.at[pl.ds(s * _RPW, _RPW)],
                    out_hbm.at[c, pl.ds(s * _RPW, _RPW)])


_spmm_call = pl.kernel(
    _spmm_body,
    out_type=jax.ShapeDtypeStruct((_NC, _NP, _D), jnp.float32),
    mesh=_mesh,
    scratch_types=(
        [pltpu.VMEM((_CH,), jnp.int32)] * 8
        + [pltpu.VMEM((_CH, _D), jnp.float32)] * 2
        + [pltpu.VMEM_SHARED((_NP, _D), jnp.float32)]
        + [pltpu.SemaphoreType.DMA] * 10
    ),
)


def _edge_body(es_hbm, ed_hbm, src2_hbm, dst2_hbm, out_hbm,
               sidx2, didx2, es0, es1, es2, es3, ed0, ed1, ed2, ed3,
               gs0, gs1, gs2, gs3, gd0, gd1, gd2, gd3):
    w = _wid()
    pltpu.sync_copy(src2_hbm.at[pl.ds(w * _NCHUNK, _NCHUNK)], sidx2)
    pltpu.sync_copy(dst2_hbm.at[pl.ds(w * _NCHUNK, _NCHUNK)], didx2)
    esb = (es0, es1, es2, es3)
    edb = (ed0, ed1, ed2, ed3)
    gss = (gs0, gs1, gs2, gs3)
    gds = (gd0, gd1, gd2, gd3)
    for b in range(_NB):
        pltpu.async_copy(es_hbm.at[sidx2.at[b]], esb[b], gss[b])
        pltpu.async_copy(ed_hbm.at[didx2.at[b]], edb[b], gds[b])

    def body(i, carry):
        for b in range(_NB):
            j = i * _NB + b
            pltpu.make_async_copy(es_hbm.at[sidx2.at[j]], esb[b], gss[b]).wait()
            pltpu.make_async_copy(ed_hbm.at[didx2.at[j]], edb[b], gds[b]).wait()
            for r in range(_CH):
                esb[b][r, :] = esb[b][r, :] + edb[b][r, :]
            pltpu.sync_copy(esb[b], out_hbm.at[pl.ds((w * _NCHUNK + j) * _CH, _CH)])

            @pl.when(j + _NB < _NCHUNK)
            def _():
                pltpu.async_copy(es_hbm.at[sidx2.at[j + _NB]], esb[b], gss[b])
                pltpu.async_copy(ed_hbm.at[didx2.at[j + _NB]], edb[b], gds[b])
        return carry

    lax.fori_loop(0, _NCHUNK // _NB, body, 0)


_edge_call = pl.kernel(
    _edge_body,
    out_type=jax.ShapeDtypeStruct((_E, 16), jnp.float32),
    mesh=_mesh,
    compiler_params=pltpu.CompilerParams(use_tc_tiling_on_sc=False),
    scratch_types=(
        [pltpu.VMEM((_NCHUNK, _CH), jnp.int32)] * 2
        + [pltpu.VMEM((_CH, 16), jnp.float32)] * 8
        + [pltpu.SemaphoreType.DMA] * 8
    ),
)


def _embed_body(ty_ref, emb_ref, wm0_ref, yinit_ref, m0_ref):
    ty = ty_ref[0, 0, :]
    oh = (ty[:, None] == lax.broadcasted_iota(jnp.int32, (_BN, _V), 1))
    oh = oh.astype(jnp.float32)
    yi = jnp.dot(oh, emb_ref[...], preferred_element_type=jnp.float32, precision=lax.Precision.HIGHEST)
    yinit_ref[...] = yi
    m0_ref[...] = jnp.dot(yi, wm0_ref[...], preferred_element_type=jnp.float32)


def _embed(ty3, emb, wm0):
    return pl.pallas_call(
        _embed_body,
        grid=(_GRID,),
        in_specs=[
            pl.BlockSpec((1, 1, _BN), lambda i: (i, 0, 0)),
            pl.BlockSpec((_V, _D), lambda i: (0, 0)),
            pl.BlockSpec((_D, _D), lambda i: (0, 0)),
        ],
        out_specs=[
            pl.BlockSpec((_BN, _D), lambda i: (i, 0)),
            pl.BlockSpec((_BN, _D), lambda i: (i, 0)),
        ],
        out_shape=[
            jax.ShapeDtypeStruct((_N, _D), jnp.float32),
            jax.ShapeDtypeStruct((_N, _D), jnp.float32),
        ],
    )(ty3, emb, wm0)


def _kv_body(x_ref, w_ref, o_ref):
    o_ref[0] = jnp.dot(x_ref[...], w_ref[0], preferred_element_type=jnp.float32)


def _kv(x, wkv):
    return pl.pallas_call(
        _kv_body,
        grid=(6,),
        in_specs=[
            pl.BlockSpec((_NX, _D), lambda i: (0, 0)),
            pl.BlockSpec((1, _D, _D), lambda i: (i, 0, 0)),
        ],
        out_specs=pl.BlockSpec((1, _NX, _D), lambda i: (i, 0, 0)),
        out_shape=jax.ShapeDtypeStruct((6, _NX, _D), jnp.float32),
    )(x, wkv)


def _block_body(agg_ref, y_ref, yi_ref, zc_ref, ws_ref, wi_ref, wet_ref,
                wq_ref, k_ref, v_ref, wp_ref, yo_ref, alpha_ref, post_ref):
    h = (agg_ref[0] + agg_ref[1]
         + jnp.dot(y_ref[...], ws_ref[...], preferred_element_type=jnp.float32)
         + jnp.dot(yi_ref[...], wi_ref[...], preferred_element_type=jnp.float32)
         + jnp.dot(zc_ref[...], wet_ref[...], preferred_element_type=jnp.float32,
                   precision=lax.Precision.HIGHEST))
    q = jnp.dot(h, wq_ref[...], preferred_element_type=jnp.float32)
    sc = lax.dot_general(q, k_ref[...], (((1,), (1,)), ((), ())),
                         preferred_element_type=jnp.float32)
    sc = sc / jnp.sqrt(jnp.float32(_D))
    mx = jnp.max(sc, axis=1, keepdims=True)
    e = jnp.exp(sc - mx)
    a = e / jnp.sum(e, axis=1, keepdims=True)
    ctx = jnp.dot(a, v_ref[...], preferred_element_type=jnp.float32)
    yo = jnp.maximum(h + ctx, 0.0)
    yo_ref[...] = yo
    alpha_ref[...] = a
    post_ref[...] = jnp.dot(yo, wp_ref[...], preferred_element_type=jnp.float32)


def _block(aggp, y, y_init, zc, ws, wi, wet, wq, k, v, wpost):
    pd = wpost.shape[1]
    return pl.pallas_call(
        _block_body,
        grid=(_GRID,),
        in_specs=[
            pl.BlockSpec((_NC, _BN, _D), lambda i: (0, i, 0)),
            pl.BlockSpec((_BN, _D), lambda i: (i, 0)),
            pl.BlockSpec((_BN, _D), lambda i: (i, 0)),
            pl.BlockSpec((_BN, _D), lambda i: (i, 0)),
            pl.BlockSpec((_D, _D), lambda i: (0, 0)),
            pl.BlockSpec((_D, _D), lambda i: (0, 0)),
            pl.BlockSpec((_D, _D), lambda i: (0, 0)),
            pl.BlockSpec((_D, _D), lambda i: (0, 0)),
            pl.BlockSpec((_NX, _D), lambda i: (0, 0)),
            pl.BlockSpec((_NX, _D), lambda i: (0, 0)),
            pl.BlockSpec((_D, pd), lambda i: (0, 0)),
        ],
        out_specs=[
            pl.BlockSpec((_BN, _D), lambda i: (i, 0)),
            pl.BlockSpec((_BN, _NX), lambda i: (i, 0)),
            pl.BlockSpec((_BN, pd), lambda i: (i, 0)),
        ],
        out_shape=[
            jax.ShapeDtypeStruct((_N, _D), jnp.float32),
            jax.ShapeDtypeStruct((_N, _NX), jnp.float32),
            jax.ShapeDtypeStruct((_N, pd), jnp.float32),
        ],
    )(aggp, y, y_init, zc, ws, wi, wet, wq, k, v, wpost)


def kernel(x, x_batch, tgt_y, tgt_edge_index, tgt_edge_type, tgt_y_batch,
           emb, Wmsg, Wself, Winit, Et, Wq, Wk, Wv, Wnode, Wedge):
    f32 = jnp.float32
    src = tgt_edge_index[0]
    dst = tgt_edge_index[1]
    zeros16 = jnp.zeros((_NP, 16), f32)
    zerosD = jnp.zeros((_NP, _D), f32)

    src2 = src.reshape(_E // _CH, _CH)
    dst2 = dst.reshape(_E // _CH, _CH)
    et2 = tgt_edge_type.reshape(_E // _CH, _CH)
    cparts = _counts_call(dst2, et2, zeros16)
    counts16 = (cparts[0] + cparts[1])[:_N]

    ty3 = tgt_y.reshape(_GRID, 1, _BN)
    y_init, m = _embed(ty3, emb, Wmsg[0])

    kv = _kv(x, jnp.concatenate([Wk, Wv], axis=0))

    zc = jnp.concatenate([counts16, jnp.zeros((_N, 112), f32)], axis=1)

    wpost2 = jnp.concatenate([
        Wnode,
        Wedge[:_D], jnp.zeros((_D, 8), f32),
        Wedge[_D:], jnp.zeros((_D, 8), f32),
        jnp.zeros((_D, 96), f32),
    ], axis=1)

    alphas = []
    y = y_init
    for i in range(3):
        wet = jnp.concatenate([Et[i], jnp.zeros((124, _D), f32)], axis=0)
        aggp = _spmm_call(m, src2, dst2, zerosD)
        wpost = wpost2 if i == 2 else Wmsg[i + 1]
        y, a, post = _block(aggp, y, y_init, zc, Wself[i], Winit[i], wet,
                            Wq[i], kv[i], kv[3 + i], wpost)
        alphas.append(a)
        m = post if i < 2 else None

    y_score = post[:, :_V]
    es16 = post[:, _V:_V + 16]
    ed16 = post[:, _V + 16:_V + 32]
    edge16 = _edge_call(es16, ed16, src2, dst2)
    y_edge_rel_score = edge16[:, :8]

    return (y_score, y_edge_rel_score, alphas[0], alphas[1], alphas[2])
